# Initial kernel scaffold; baseline (speedup 1.0000x reference)
import jax
import jax.numpy as jnp
from jax.experimental import pallas as pl


def kernel(input, weight):
    raise NotImplementedError("write your pallas kernel here")


# speedup vs baseline: 4.4970x; 4.4970x over previous
import functools

import jax
import jax.numpy as jnp
import numpy as np
from jax import lax
from jax.experimental import pallas as pl
from jax.experimental.pallas import tpu as pltpu
from jax.experimental.pallas import tpu_sc as plsc
from jax.scipy.special import ndtri

D = 64
NW = 32
C = 256
KC = C // 128
G = C // 16
FRAC = np.float32(0.999 * 63 - 62)


def _sc_body(idx_hbm, w_hbm, midlut_hbm, flut_hbm, out_hbm,
             idx_v, rows_v, out_v, midlut_v, flut_v, sem, nchunk):
    wid = lax.axis_index("s") * 2 + lax.axis_index("c")
    pltpu.sync_copy(midlut_hbm, midlut_v)
    pltpu.sync_copy(flut_hbm, flut_v)
    iota16 = lax.iota(jnp.int32, 16)
    perms = [iota16 ^ (1 << b) for b in range(4)]
    rpw = nchunk * C

    @pl.loop(0, nchunk)
    def _chunk(g):
        row0 = wid * rpw + g * C
        pltpu.sync_copy(idx_hbm.at[pl.ds(wid * (rpw // 128) + g * KC, KC)],
                        idx_v)
        cps = [pltpu.async_copy(w_hbm.at[idx_v.at[j]],
                                rows_v.at[pl.ds(j * 128, 128)], sem)
               for j in range(KC)]
        for cp in cps:
            cp.wait()

        @pl.loop(0, C, unroll=2)
        def _row(r):
            v = [rows_v[r, pl.ds(16 * k, 16)] for k in range(4)]
            a = [jnp.abs(x) for x in v]
            s1 = jnp.maximum(a[0], a[1])
            t1 = jnp.minimum(a[0], a[1])
            s2 = jnp.maximum(a[2], a[3])
            t2 = jnp.minimum(a[2], a[3])
            m1 = jnp.maximum(s1, s2)
            m2 = jnp.maximum(jnp.minimum(s1, s2), jnp.maximum(t1, t2))
            for p in perms:
                pm1 = m1.at[p].get(mode="promise_in_bounds")
                pm2 = m2.at[p].get(mode="promise_in_bounds")
                m2 = jnp.maximum(jnp.minimum(m1, pm1),
                                 jnp.where(m1 >= pm1, m2, pm2))
                m1 = jnp.maximum(m1, pm1)
            amax = jnp.maximum(m2 + FRAC * (m1 - m2), np.float32(1e-8))
            inv = np.float32(1.0) / amax
            namax = -amax
            for k in range(4):
                xn = jnp.minimum(jnp.maximum(v[k], namax), amax) * inv
                u = jnp.minimum(((xn + np.float32(1.0)) * np.float32(128.0))
                                .astype(jnp.int32), 255)
                mv = plsc.load_gather(midlut_v, [u])
                u2 = u + u + jnp.where(xn > mv, 1, 0)
                q = plsc.load_gather(flut_v, [u2])
                out_v[r, pl.ds(16 * k, 16)] = q * amax

        pltpu.sync_copy(out_v, out_hbm.at[pl.ds(row0, C)])


@jax.jit
def _axs_embed(idx2d, weight, midlut, flut):
    nrows = idx2d.shape[0] * 128
    nchunk = nrows // (NW * C)
    body = functools.partial(_sc_body, nchunk=nchunk)
    f = pl.kernel(
        body,
        out_type=jax.ShapeDtypeStruct((nrows, D), jnp.float32),
        mesh=plsc.VectorSubcoreMesh(core_axis_name="c", subcore_axis_name="s",
                                    num_cores=2, num_subcores=16),
        scratch_types=[
            pltpu.VMEM((KC, 128), jnp.int32),
            pltpu.VMEM((C, D), jnp.float32),
            pltpu.VMEM((C, D), jnp.float32),
            pltpu.VMEM((256,), jnp.float32),
            pltpu.VMEM((512,), jnp.float32),
            pltpu.SemaphoreType.DMA,
        ],
        compiler_params=pltpu.CompilerParams(needs_layout_passes=False, use_tc_tiling_on_sc=False),
    )
    return f(idx2d, weight, midlut, flut)


def kernel(input, weight):
    nrows = input.shape[0] * input.shape[1]
    idx2d = input.reshape(nrows // 128, 128)
    probs = (jnp.arange(32, dtype=jnp.float32) + 0.5) / 32
    lv = ndtri(probs)
    lv = (lv / jnp.max(jnp.abs(lv))).astype(jnp.float32)
    mids = (lv[:-1] + lv[1:]) * np.float32(0.5)
    midpad = jnp.concatenate([mids, jnp.full((1,), 2.0, jnp.float32)])
    edges = jnp.arange(256, dtype=jnp.float32) / np.float32(128.0) - 1
    lut = jnp.sum(mids[None, :] < edges[:, None], axis=1).astype(jnp.int32)
    midlut = midpad[lut]
    flut = lv[jnp.minimum(lut[:, None] + jnp.arange(2)[None, :], 31)].reshape(512)
    out = _axs_embed(idx2d, weight, midlut, flut)
    return out.reshape(input.shape[0], input.shape[1], D)
